# Initial kernel scaffold; baseline (speedup 1.0000x reference)
import jax
import jax.numpy as jnp
from jax.experimental import pallas as pl


def kernel(h, e, p, edge_index, batch, params):
    raise NotImplementedError("write your pallas kernel here")


# speedup vs baseline: 5.3709x; 5.3709x over previous
import functools

import jax
import jax.numpy as jnp
from jax import lax
from jax.experimental import pallas as pl
from jax.experimental.pallas import tpu as pltpu
from jax.experimental.pallas import tpu_sc as plsc

NC = 2
NS = 16
NW = NC * NS
CH = 128
H = 128
EF = 16
BLK = 1000


def _mm(x, w):
    return lax.dot_general(x, w, (((1,), (0,)), ((), ())),
                           preferred_element_type=jnp.float32,
                           precision=lax.Precision.HIGHEST)


def _spmem_chunks(zrows):
    out = []
    off = 0
    while off < zrows:
        sz = min(CH, zrows - off)
        out.append((off, sz))
        off += sz
    return out


@functools.lru_cache(maxsize=None)
def _make_scatter(npad, cpt, d):
    zrows = npad // NS
    mesh = plsc.VectorSubcoreMesh(core_axis_name="c", subcore_axis_name="s")

    @functools.partial(
        pl.kernel,
        out_type=jax.ShapeDtypeStruct((NC, npad, d), jnp.float32),
        mesh=mesh,
        scratch_types=[
            pltpu.VMEM((cpt, CH), jnp.int32),
            pltpu.VMEM((cpt, CH), jnp.int32),
            pltpu.VMEM((CH, d), jnp.float32),
            pltpu.VMEM_SHARED((npad, d), jnp.float32),
            pltpu.SemaphoreType.DMA,
        ],
    )
    def scatter_k(table, send_h, rec_h, zeros_h, out,
                  send_v, rec_v, rows, acc, sem):
        cid = lax.axis_index("c")
        sid = lax.axis_index("s")
        wid = sid * NC + cid
        zb = sid * zrows
        pltpu.sync_copy(zeros_h, rows)
        for off, sz in _spmem_chunks(zrows):
            pltpu.sync_copy(rows.at[pl.ds(0, sz)], acc.at[pl.ds(zb + off, sz)])
        pltpu.sync_copy(send_h.at[wid], send_v)
        pltpu.sync_copy(rec_h.at[wid], rec_v)
        plsc.subcore_barrier()

        def step(j, carry):
            pltpu.sync_copy(table.at[send_v.at[j]], rows)
            pltpu.sync_copy(rows, acc.at[rec_v.at[j]], add=True)
            return carry

        lax.fori_loop(0, cpt, step, 0)
        plsc.subcore_barrier()
        for off, sz in _spmem_chunks(zrows):
            pltpu.sync_copy(acc.at[pl.ds(zb + off, sz)], rows.at[pl.ds(0, sz)])
            pltpu.sync_copy(rows.at[pl.ds(0, sz)], out.at[cid, pl.ds(zb + off, sz)])

    return scatter_k


def _row(j):
    return (j, 0)


def _w2(j):
    return (0, 0)


def _w3(j):
    return (0, 0, 0)


def _part3(j):
    return (0, j, 0)


def _embed(h_in, p_in, wh, bh, wp, bp, w1, w2):
    n = h_in.shape[0]

    def body(h_ref, p_ref, wh_r, bh_r, wp_r, bp_r, w1_r, w2_r,
             h0_ref, p0_ref, u0_ref):
        h0 = _mm(h_ref[...], wh_r[...]) + bh_r[...]
        p0 = _mm(p_ref[...], wp_r[...]) + bp_r[...]
        h0_ref[...] = h0
        p0_ref[...] = p0
        u0_ref[...] = _mm(h0, w1_r[...]) + _mm(p0, w2_r[...])

    return pl.pallas_call(
        body,
        grid=(n // BLK,),
        in_specs=[
            pl.BlockSpec((BLK, H), _row),
            pl.BlockSpec((BLK, EF), _row),
            pl.BlockSpec((H, H), _w2),
            pl.BlockSpec((1, H), _w2),
            pl.BlockSpec((EF, H), _w2),
            pl.BlockSpec((1, H), _w2),
            pl.BlockSpec((H, H), _w2),
            pl.BlockSpec((H, H), _w2),
        ],
        out_specs=[pl.BlockSpec((BLK, H), _row)] * 3,
        out_shape=[jax.ShapeDtypeStruct((n, H), jnp.float32)] * 3,
    )(h_in, p_in, wh, bh, wp, bp, w1, w2)


def _stage_b(h, p, s, r, hp, qd, wb, qc, b3):
    n = h.shape[0]

    def body(h_ref, p_ref, s_ref, r_ref, hp_ref, qd_ref,
             wb_r, qc_r, b3_r, hn_ref, sn_ref, rn_ref, vs_ref, rp_ref):
        h_ = h_ref[...]
        p_ = p_ref[...]
        s_ = s_ref[...]
        r_ = r_ref[...]
        part = hp_ref[0] + hp_ref[1]
        qdsum = qd_ref[0] + qd_ref[1]
        qsum = qdsum[:, 0:EF]
        deg = qdsum[:, EF:EF + 1]
        w3 = wb_r[0]
        w4 = wb_r[1]
        w5 = wb_r[2]
        wg1 = wb_r[3]
        wg2 = wb_r[4]
        v1 = wb_r[5]
        v2 = wb_r[6]
        v3 = wb_r[7]
        u1 = wb_r[8]
        u2 = wb_r[9]
        u3 = wb_r[10]
        bias_h = b3_r[0:1]
        bias_p = b3_r[1:2]
        bg = b3_r[2:3]
        agg = part + deg * (_mm(h_, w3) + _mm(p_, w4) + _mm(r_, w5) + bias_h) \
            + _mm(qsum, qc_r[0])
        h_new = _mm(h_, wg1) + _mm(agg, wg2) + bg
        s_new = _mm(h_new, v1) + _mm(s_, v3)
        r_new = _mm(h_new, v2) + _mm(r_, v3)
        hn_ref[...] = h_new
        sn_ref[...] = s_new
        rn_ref[...] = r_new
        vs_ref[...] = _mm(p_, u1) + _mm(s_new, u3)
        rp_ref[...] = deg * (_mm(p_, u2) + _mm(r_new, u3) + bias_p) \
            + _mm(qsum, qc_r[1])

    return pl.pallas_call(
        body,
        grid=(n // BLK,),
        in_specs=[
            pl.BlockSpec((BLK, H), _row),
            pl.BlockSpec((BLK, H), _row),
            pl.BlockSpec((BLK, H), _row),
            pl.BlockSpec((BLK, H), _row),
            pl.BlockSpec((NC, BLK, H), _part3),
            pl.BlockSpec((NC, BLK, H), _part3),
            pl.BlockSpec((11, H, H), _w3),
            pl.BlockSpec((2, EF, H), _w3),
            pl.BlockSpec((3, H), _w2),
        ],
        out_specs=[pl.BlockSpec((BLK, H), _row)] * 5,
        out_shape=[jax.ShapeDtypeStruct((n, H), jnp.float32)] * 5,
    )(h, p, s, r, hp, qd, wb, qc, b3)


def _stage_ca(p, pp, restp, h_new, s_new, wc, bpg):
    n = p.shape[0]

    def body(p_ref, pp_ref, rp_ref, hn_ref, sn_ref, wc_r, bpg_r,
             pn_ref, un_ref):
        aggp = pp_ref[0] + pp_ref[1] + rp_ref[...]
        p_new = _mm(p_ref[...], wc_r[0]) + _mm(aggp, wc_r[1]) + bpg_r[...]
        pn_ref[...] = p_new
        un_ref[...] = _mm(hn_ref[...], wc_r[2]) + _mm(p_new, wc_r[3]) \
            + _mm(sn_ref[...], wc_r[4])

    return pl.pallas_call(
        body,
        grid=(n // BLK,),
        in_specs=[
            pl.BlockSpec((BLK, H), _row),
            pl.BlockSpec((NC, BLK, H), _part3),
            pl.BlockSpec((BLK, H), _row),
            pl.BlockSpec((BLK, H), _row),
            pl.BlockSpec((BLK, H), _row),
            pl.BlockSpec((5, H, H), _w3),
            pl.BlockSpec((1, H), _w2),
        ],
        out_specs=[pl.BlockSpec((BLK, H), _row)] * 2,
        out_shape=[jax.ShapeDtypeStruct((n, H), jnp.float32)] * 2,
    )(p, pp, restp, h_new, s_new, wc, bpg)


def _stage_cf(p, pp, restp, wf, bpg):
    n = p.shape[0]

    def body(p_ref, pp_ref, rp_ref, wf_r, bpg_r, pn_ref):
        aggp = pp_ref[0] + pp_ref[1] + rp_ref[...]
        pn_ref[...] = _mm(p_ref[...], wf_r[0]) + _mm(aggp, wf_r[1]) + bpg_r[...]

    return pl.pallas_call(
        body,
        grid=(n // BLK,),
        in_specs=[
            pl.BlockSpec((BLK, H), _row),
            pl.BlockSpec((NC, BLK, H), _part3),
            pl.BlockSpec((BLK, H), _row),
            pl.BlockSpec((2, H, H), _w3),
            pl.BlockSpec((1, H), _w2),
        ],
        out_specs=pl.BlockSpec((BLK, H), _row),
        out_shape=jax.ShapeDtypeStruct((n, H), jnp.float32),
    )(p, pp, restp, wf, bpg)


def _readout(h, p, batch_r, g, w0h, w0p, b0, w1, b1, w2p, b2p):
    n = h.shape[0]
    nb = n // BLK

    def body(h_ref, p_ref, b_ref, w0h_r, w0p_r, b0_r, w1_r, b1_r,
             w2_r, b2_r, out_ref, acc_h, acc_p):
        j = pl.program_id(0)

        @pl.when(j == 0)
        def _():
            acc_h[...] = jnp.zeros((g, H), jnp.float32)
            acc_p[...] = jnp.zeros((g, H), jnp.float32)

        ids = b_ref[0, 0, :]
        iota = lax.broadcasted_iota(jnp.int32, (g, BLK), 0)
        oh = (ids[None, :] == iota).astype(jnp.float32)
        acc_h[...] += _mm(oh, h_ref[...])
        acc_p[...] += _mm(oh, p_ref[...])

        @pl.when(j == nb - 1)
        def _():
            x = jax.nn.relu(_mm(acc_h[...], w0h_r[...])
                            + _mm(acc_p[...], w0p_r[...]) + b0_r[...])
            x = jax.nn.relu(_mm(x, w1_r[...]) + b1_r[...])
            out_ref[...] = _mm(x, w2_r[...]) + b2_r[...]

    return pl.pallas_call(
        body,
        grid=(nb,),
        in_specs=[
            pl.BlockSpec((BLK, H), _row),
            pl.BlockSpec((BLK, H), _row),
            pl.BlockSpec((1, 1, BLK), lambda j: (j, 0, 0)),
            pl.BlockSpec((H, H), _w2),
            pl.BlockSpec((H, H), _w2),
            pl.BlockSpec((1, H), _w2),
            pl.BlockSpec((H, 64), _w2),
            pl.BlockSpec((1, 64), _w2),
            pl.BlockSpec((64, H), _w2),
            pl.BlockSpec((1, H), _w2),
        ],
        out_specs=pl.BlockSpec((g, H), _w2),
        out_shape=jax.ShapeDtypeStruct((g, H), jnp.float32),
        scratch_shapes=[pltpu.VMEM((g, H), jnp.float32),
                        pltpu.VMEM((g, H), jnp.float32)],
    )(h, p, batch_r, w0h, w0p, b0, w1, b1, w2p, b2p)


def kernel(h, e, p, edge_index, batch, params):
    n = h.shape[0]
    n_edges = e.shape[0]
    g = 64
    npad = ((n + 1 + NS * 8 - 1) // (NS * 8)) * (NS * 8)
    cpt = -(-n_edges // (NW * CH))
    epad = NW * CH * cpt - n_edges

    send = edge_index[0]
    rec = edge_index[1]
    send_r = jnp.concatenate(
        [send, jnp.zeros((epad,), jnp.int32)]).reshape(NW, cpt, CH)
    rec_r = jnp.concatenate(
        [rec, jnp.full((epad,), n, jnp.int32)]).reshape(NW, cpt, CH)
    t128 = jnp.concatenate(
        [e, jnp.ones((n_edges, 1), jnp.float32),
         jnp.zeros((n_edges, H - EF - 1), jnp.float32)], axis=1)
    t128 = jnp.concatenate([t128, jnp.zeros((epad, H), jnp.float32)])
    eid_r = jnp.arange(n_edges + epad, dtype=jnp.int32).reshape(NW, cpt, CH)
    zeros_h = jnp.zeros((CH, H), jnp.float32)
    batch_r = batch.reshape(n // BLK, 1, BLK)

    pp = params
    wb_l, qc_l, b3_l, wc_l = [], [], [], []
    em = pp["e_embed"]["W"]
    ec = pp["e_embed"]["b"]
    layers = pp["layers"]
    hu_w = [lp["h_update"]["W"] for lp in layers]
    for li, lp in enumerate(layers):
        whu, b1 = lp["h_update"]["W"], lp["h_update"]["b"]
        w1, w2, w3, w4, w5 = (whu[0:H], whu[H:2 * H], whu[2 * H:3 * H],
                              whu[3 * H:4 * H], whu[4 * H:5 * H])
        veu, b2 = lp["e_update"]["W"], lp["e_update"]["b"]
        v1, v2, v3 = veu[0:H], veu[H:2 * H], veu[2 * H:3 * H]
        upu, b3v = lp["p_update"]["W"], lp["p_update"]["b"]
        u1, u2, u3 = upu[0:H], upu[H:2 * H], upu[2 * H:3 * H]
        wg, bg = lp["h_message_agg_update"]["W"], lp["h_message_agg_update"]["b"]
        wpa, bpa = lp["p_message_agg_update"]["W"], lp["p_message_agg_update"]["b"]
        em_new = em @ v3
        ec_new = ec @ v3 + b2
        wb_l.append(jnp.stack([w3, w4, w5, wg[0:H], wg[H:2 * H],
                               v1, v2, v3, u1, u2, u3]))
        qc_l.append(jnp.stack([em @ w5, em_new @ u3]))
        b3_l.append(jnp.stack([ec @ w5 + b1, ec_new @ u3 + b3v, bg]))
        if li < len(layers) - 1:
            nxt = hu_w[li + 1]
            wc_l.append(jnp.stack([wpa[0:H], wpa[H:2 * H],
                                   nxt[0:H], nxt[H:2 * H], nxt[4 * H:5 * H]]))
        else:
            wf = jnp.stack([wpa[0:H], wpa[H:2 * H]])
        em, ec = em_new, ec_new
    bpg_l = [lp["p_message_agg_update"]["b"].reshape(1, H) for lp in layers]

    r0 = pp["readout"]
    w0h, w0p = r0[0]["W"][0:H], r0[0]["W"][H:2 * H]
    b0 = r0[0]["b"].reshape(1, H)
    w1r = r0[1]["W"]
    b1r = r0[1]["b"].reshape(1, 64)
    w2p = jnp.zeros((64, H), jnp.float32).at[:, 0].set(r0[2]["W"][:, 0])
    b2p = jnp.zeros((1, H), jnp.float32).at[0, 0].set(r0[2]["b"][0])

    scatter = _make_scatter(npad, cpt, H)

    h0, p0, u0 = _embed(h, p, pp["h_embed"]["W"], pp["h_embed"]["b"].reshape(1, H),
                        pp["p_embed"]["W"], pp["p_embed"]["b"].reshape(1, H),
                        hu_w[0][0:H], hu_w[0][H:2 * H])
    qd = scatter(t128, eid_r, rec_r, zeros_h)

    hcur, pcur = h0, p0
    s = jnp.zeros((n, H), jnp.float32)
    r = jnp.zeros((n, H), jnp.float32)
    ucur = u0
    for li in range(len(layers)):
        hp = scatter(ucur, send_r, rec_r, zeros_h)
        h_new, s_new, r_new, vsc, restp = _stage_b(
            hcur, pcur, s, r, hp, qd, wb_l[li], qc_l[li], b3_l[li])
        ppart = scatter(vsc, send_r, rec_r, zeros_h)
        if li < len(layers) - 1:
            p_new, ucur = _stage_ca(pcur, ppart, restp, h_new, s_new,
                                    wc_l[li], bpg_l[li])
        else:
            p_new = _stage_cf(pcur, ppart, restp, wf, bpg_l[li])
        hcur, pcur, s, r = h_new, p_new, s_new, r_new

    ro = _readout(hcur, pcur, batch_r, g, w0h, w0p, b0, w1r, b1r, w2p, b2p)
    return ro[:, 0], pcur
